# speedup vs baseline: 1.5987x; 1.5987x over previous
import functools

import jax
import jax.numpy as jnp
from jax import lax
from jax.experimental import pallas as pl
from jax.experimental.pallas import tpu as pltpu, tpu_sc as plsc

_VOCAB = 50432
_D = 6144
_B = 8192

_NC = 2
_NS = 16
_NW = _NC * _NS
_BPW = _B // _NW
_CH = 8
_NCHUNK = _BPW // _CH

_mesh = plsc.VectorSubcoreMesh(core_axis_name="c", subcore_axis_name="s")


@functools.partial(
    pl.kernel,
    out_type=jax.ShapeDtypeStruct((_B, _D), jnp.float32),
    mesh=_mesh,
    scratch_types=[
        pltpu.VMEM((_BPW,), jnp.int32),
        pltpu.VMEM((_CH, _D), jnp.float32),
        pltpu.SemaphoreType.DMA,
    ],
)
def _lookup(w_hbm, x_hbm, out_hbm, idx_v, buf_v, gsem):
    wid = lax.axis_index("s") * _NC + lax.axis_index("c")
    base = wid * _BPW
    pltpu.sync_copy(x_hbm.at[pl.ds(base, _BPW)], idx_v)

    def body(c, carry):
        off = pl.multiple_of(c * _CH, _CH)
        pltpu.async_copy(w_hbm.at[idx_v.at[pl.ds(off, _CH)]], buf_v, gsem).wait()
        pltpu.sync_copy(buf_v, out_hbm.at[pl.ds(base + off, _CH)])
        return carry

    lax.fori_loop(0, _NCHUNK, body, 0)


def kernel(x, W):
    flat = _lookup(W, x.reshape(-1))
    return flat.reshape(x.shape + (W.shape[1],))


# speedup vs baseline: 1.7925x; 1.1212x over previous
import functools

import jax
import jax.numpy as jnp
from jax import lax
from jax.experimental import pallas as pl
from jax.experimental.pallas import tpu as pltpu, tpu_sc as plsc

_VOCAB = 50432
_D = 6144
_B = 8192

_NC = 2
_NS = 16
_NW = _NC * _NS
_BPW = _B // _NW
_CH = 8
_NCHUNK = _BPW // _CH
_NBUF = 2

_mesh = plsc.VectorSubcoreMesh(core_axis_name="c", subcore_axis_name="s")


@functools.partial(
    pl.kernel,
    out_type=jax.ShapeDtypeStruct((_B, _D), jnp.float32),
    mesh=_mesh,
    scratch_types=[
        pltpu.VMEM((_NCHUNK, _CH), jnp.int32),
        pltpu.VMEM((_NBUF, _CH, _D), jnp.float32),
        pltpu.SemaphoreType.DMA,
        pltpu.SemaphoreType.DMA,
        pltpu.SemaphoreType.DMA,
        pltpu.SemaphoreType.DMA,
    ],
)
def _lookup(w_hbm, x_hbm, out_hbm, idx_v, buf_v, g0, g1, p0, p1):
    gs = (g0, g1)
    ps = (p0, p1)
    wid = lax.axis_index("s") * _NC + lax.axis_index("c")
    base = wid * _BPW
    pltpu.sync_copy(x_hbm.at[wid], idx_v)

    def wait_gather(b):
        pltpu.make_async_copy(w_hbm.at[pl.ds(0, _CH)], buf_v.at[b], gs[b]).wait()

    def start_write(b, c):
        off = pl.multiple_of(base + c * _CH, _CH)
        pltpu.async_copy(buf_v.at[b], out_hbm.at[pl.ds(off, _CH)], ps[b])

    def wait_write(b):
        pltpu.make_async_copy(
            buf_v.at[b], out_hbm.at[pl.ds(base, _CH)], ps[b]
        ).wait()

    def start_gather(b, c):
        pltpu.async_copy(w_hbm.at[idx_v.at[c]], buf_v.at[b], gs[b])

    for b in range(_NBUF):
        start_gather(b, b)

    def body(g, carry):
        for b in range(_NBUF):
            c = g * _NBUF + b
            wait_gather(b)
            start_write(b, c)
            wait_write(b)
            start_gather(b, c + _NBUF)
        return carry

    lax.fori_loop(0, _NCHUNK // _NBUF - 1, body, 0)

    for b in range(_NBUF):
        wait_gather(b)
        start_write(b, _NCHUNK - _NBUF + b)
    for b in range(_NBUF):
        wait_write(b)


def kernel(x, W):
    flat = _lookup(W, x.reshape(_NW, _NCHUNK, _CH))
    return flat.reshape(x.shape + (W.shape[1],))
